# Initial kernel scaffold; baseline (speedup 1.0000x reference)
import jax
import jax.numpy as jnp
from jax.experimental import pallas as pl


def kernel(x, edge_index, edge_attr, batch, W_init, b_init, W_edge, b_edge, bias, W_ih, W_hh, b_ih, b_hh, W_fi, b_fi, W_fj, b_fj, W_score, b_score, W_fc, b_fc):
    raise NotImplementedError("write your pallas kernel here")


# speedup vs baseline: 13.2731x; 13.2731x over previous
import functools

import jax
import jax.numpy as jnp
from jax import lax
from jax.experimental import pallas as pl
from jax.experimental.pallas import tpu as pltpu
from jax.experimental.pallas import tpu_sc as plsc

N = 10000
E = 320000
F = 128
H = 128
T = 3
B = 16
ITER = 3

ROWS = 400
NBLK = N // ROWS

NC = 2
NS = 16
NW = NC * NS
CH = 128
NCH_MAIN = E // (NW * CH)
NCH = NCH_MAIN + 1
NXW = (E - NW * NCH_MAIN * CH) // CH
NPAD = 10240
STRIP = NPAD // NS

_pallas_call = pl.pallas_call
_f32 = jnp.float32


def _dot(a, b):
    return jnp.dot(a, b, preferred_element_type=_f32)


def _stage_init_body(x_ref, wi_ref, bi_ref, we_ref, be_ref, h_ref, g_ref):
    xb = x_ref[...]
    hb = _dot(xb, wi_ref[...]) + bi_ref[...]
    h_ref[...] = hb
    for t in range(T):
        g_ref[t] = _dot(hb, we_ref[t]) + be_ref[t]


def _stage_init(x, w_init, b_init2, w_edge, b_edge2):
    return _pallas_call(
        _stage_init_body,
        grid=(NBLK,),
        in_specs=[
            pl.BlockSpec((ROWS, F), lambda i: (i, 0)),
            pl.BlockSpec((F, H), lambda i: (0, 0)),
            pl.BlockSpec((1, H), lambda i: (0, 0)),
            pl.BlockSpec((T, H, H), lambda i: (0, 0, 0)),
            pl.BlockSpec((T, H), lambda i: (0, 0)),
        ],
        out_specs=[
            pl.BlockSpec((ROWS, H), lambda i: (i, 0)),
            pl.BlockSpec((T, ROWS, H), lambda i: (0, i, 0)),
        ],
        out_shape=[
            jax.ShapeDtypeStruct((N, H), _f32),
            jax.ShapeDtypeStruct((T, N, H), _f32),
        ],
    )(x, w_init, b_init2, w_edge, b_edge2)


def _stage_gru_body(emit_g, mp_ref, h_ref, wih_ref, whh_ref, bih_ref,
                    bhh_ref, we_ref, be_ref, hn_ref, *maybe_g):
    m = mp_ref[0] + mp_ref[1]
    h = h_ref[...]
    gi = _dot(m, wih_ref[...]) + bih_ref[...]
    gh = _dot(h, whh_ref[...]) + bhh_ref[...]
    r = jax.nn.sigmoid(gi[:, 0:H] + gh[:, 0:H])
    z = jax.nn.sigmoid(gi[:, H:2 * H] + gh[:, H:2 * H])
    n = jnp.tanh(gi[:, 2 * H:] + r * gh[:, 2 * H:])
    hn = (1.0 - z) * n + z * h
    hn_ref[...] = hn
    if emit_g:
        g_ref = maybe_g[0]
        for t in range(T):
            g_ref[t] = _dot(hn, we_ref[t]) + be_ref[t]


def _stage_gru(msg_partial, h, w_ih, w_hh, b_ih2, b_hh2, w_edge, b_edge2,
               emit_g):
    out_specs = [pl.BlockSpec((ROWS, H), lambda i: (i, 0))]
    out_shape = [jax.ShapeDtypeStruct((N, H), _f32)]
    if emit_g:
        out_specs.append(pl.BlockSpec((T, ROWS, H), lambda i: (0, i, 0)))
        out_shape.append(jax.ShapeDtypeStruct((T, N, H), _f32))
    return _pallas_call(
        functools.partial(_stage_gru_body, emit_g),
        grid=(NBLK,),
        in_specs=[
            pl.BlockSpec((NC, ROWS, H), lambda i: (0, i, 0)),
            pl.BlockSpec((ROWS, H), lambda i: (i, 0)),
            pl.BlockSpec((H, 3 * H), lambda i: (0, 0)),
            pl.BlockSpec((H, 3 * H), lambda i: (0, 0)),
            pl.BlockSpec((1, 3 * H), lambda i: (0, 0)),
            pl.BlockSpec((1, 3 * H), lambda i: (0, 0)),
            pl.BlockSpec((T, H, H), lambda i: (0, 0, 0)),
            pl.BlockSpec((T, H), lambda i: (0, 0)),
        ],
        out_specs=out_specs,
        out_shape=out_shape,
    )(msg_partial, h, w_ih, w_hh, b_ih2, b_hh2, w_edge, b_edge2)


def _stage_readout_body(h_ref, x_ref, b_ref, wfih_ref, wfix_ref, bfi_ref,
                        wfjh_ref, wfjx_ref, bfj_ref, ws_ref, bs_ref,
                        wfc_ref, bfc_ref, o_ref, acc_ref):
    i = pl.program_id(0)

    @pl.when(i == 0)
    def _():
        acc_ref[...] = jnp.zeros((B, H), _f32)

    hb = h_ref[...]
    xb = x_ref[...]
    fi = _dot(hb, wfih_ref[...]) + _dot(xb, wfix_ref[...]) + bfi_ref[...]
    fj = jnp.tanh(_dot(hb, wfjh_ref[...]) + _dot(xb, wfjx_ref[...])
                  + bfj_ref[...])
    srow = jnp.sum(fi * ws_ref[...], axis=1, keepdims=True) + bs_ref[0, 0]
    wgt = jax.nn.sigmoid(srow) * fj
    bid = b_ref[0, 0, :]
    onehot = (lax.broadcasted_iota(jnp.int32, (B, ROWS), 0)
              == bid[None, :]).astype(_f32)
    acc_ref[...] += _dot(onehot, wgt)

    @pl.when(i == NBLK - 1)
    def _():
        o_ref[...] = _dot(jnp.tanh(acc_ref[...]), wfc_ref[...]) + bfc_ref[...]


def _stage_readout(h, x, batch3, w_fi_h, w_fi_x, b_fi2, w_fj_h, w_fj_x,
                   b_fj2, ws_row, bs11, w_fc, b_fc2):
    return _pallas_call(
        _stage_readout_body,
        grid=(NBLK,),
        in_specs=[
            pl.BlockSpec((ROWS, H), lambda i: (i, 0)),
            pl.BlockSpec((ROWS, F), lambda i: (i, 0)),
            pl.BlockSpec((1, 1, ROWS), lambda i: (i, 0, 0)),
            pl.BlockSpec((H, H), lambda i: (0, 0)),
            pl.BlockSpec((F, H), lambda i: (0, 0)),
            pl.BlockSpec((1, H), lambda i: (0, 0)),
            pl.BlockSpec((H, H), lambda i: (0, 0)),
            pl.BlockSpec((F, H), lambda i: (0, 0)),
            pl.BlockSpec((1, H), lambda i: (0, 0)),
            pl.BlockSpec((1, H), lambda i: (0, 0)),
            pl.BlockSpec((1, 1), lambda i: (0, 0)),
            pl.BlockSpec((H, H), lambda i: (0, 0)),
            pl.BlockSpec((1, H), lambda i: (0, 0)),
        ],
        out_specs=pl.BlockSpec((B, H), lambda i: (0, 0)),
        out_shape=jax.ShapeDtypeStruct((B, H), _f32),
        scratch_shapes=[pltpu.VMEM((B, H), _f32)],
        compiler_params=pltpu.CompilerParams(
            dimension_semantics=("arbitrary",)),
    )(h, x, batch3, w_fi_h, w_fi_x, b_fi2, w_fj_h, w_fj_x, b_fj2, ws_row,
      bs11, w_fc, b_fc2)


def _sc_body(g_hbm, gidx_hbm, dst_hbm, zeros_hbm, out_hbm,
             gidx_v, dst_v, rows_v, acc_sh):
    cid = lax.axis_index("c")
    sid = lax.axis_index("s")
    wid = sid * NC + cid

    pltpu.sync_copy(zeros_hbm, acc_sh.at[pl.ds(sid * STRIP, STRIP)])

    pltpu.sync_copy(gidx_hbm.at[pl.ds(wid * (NCH * CH), NCH * CH)], gidx_v)
    pltpu.sync_copy(dst_hbm.at[wid], dst_v)
    plsc.subcore_barrier()

    @pl.loop(0, NCH_MAIN)
    def _(j):
        pltpu.sync_copy(g_hbm.at[gidx_v.at[pl.ds(j * CH, CH)]], rows_v)
        pltpu.sync_copy(rows_v, acc_sh.at[dst_v.at[j]], add=True)

    @pl.when(wid < NXW)
    def _():
        pltpu.sync_copy(g_hbm.at[gidx_v.at[pl.ds(NCH_MAIN * CH, CH)]], rows_v)
        pltpu.sync_copy(rows_v, acc_sh.at[dst_v.at[NCH_MAIN]], add=True)

    plsc.subcore_barrier()
    pltpu.sync_copy(acc_sh.at[pl.ds(sid * STRIP, STRIP)],
                    out_hbm.at[cid, pl.ds(sid * STRIP, STRIP)])


def _sc_messages(g_table, gidx_r, dst_r, zeros):
    mesh = plsc.VectorSubcoreMesh(core_axis_name="c", subcore_axis_name="s")
    k = pl.kernel(
        _sc_body,
        mesh=mesh,
        out_type=jax.ShapeDtypeStruct((NC, NPAD, H), _f32),
        scratch_types=[
            pltpu.VMEM((NCH * CH,), jnp.int32),
            pltpu.VMEM((NCH, CH), jnp.int32),
            pltpu.VMEM((CH, H), _f32),
            pltpu.VMEM_SHARED((NPAD, H), _f32),
        ],
    )
    return k(g_table, gidx_r, dst_r, zeros)


def kernel(x, edge_index, edge_attr, batch, W_init, b_init, W_edge, b_edge,
           bias, W_ih, W_hh, b_ih, b_hh, W_fi, b_fi, W_fj, b_fj, W_score,
           b_score, W_fc, b_fc):
    src = edge_index[0]
    dst = edge_index[1]

    gidx = edge_attr.astype(jnp.int32) * N + src
    e_main = NW * NCH_MAIN * CH
    gidx_main = gidx[:e_main].reshape(NW, NCH_MAIN * CH)
    gidx_x = jnp.zeros((NW, CH), jnp.int32).at[:NXW].set(
        gidx[e_main:].reshape(NXW, CH))
    gidx_r = jnp.concatenate([gidx_main, gidx_x], axis=1).reshape(-1)
    dst_main = dst[:e_main].reshape(NW, NCH_MAIN, CH)
    dst_x = jnp.zeros((NW, 1, CH), jnp.int32).at[:NXW].set(
        dst[e_main:].reshape(NXW, 1, CH))
    dst_r = jnp.concatenate([dst_main, dst_x], axis=1)
    zeros = jnp.zeros((STRIP, H), _f32)

    b_init2 = b_init.reshape(1, H)
    b_edge2 = b_edge + bias[None, :]
    b_ih2 = b_ih.reshape(1, 3 * H)
    b_hh2 = b_hh.reshape(1, 3 * H)

    h, g = _stage_init(x, W_init, b_init2, W_edge, b_edge2)
    for it in range(ITER):
        msg_partial = _sc_messages(g.reshape(T * N, H), gidx_r, dst_r, zeros)
        emit_g = it < ITER - 1
        outs = _stage_gru(msg_partial, h, W_ih, W_hh, b_ih2, b_hh2, W_edge,
                          b_edge2, emit_g)
        if emit_g:
            h, g = outs
        else:
            h, = outs

    batch3 = batch.reshape(NBLK, 1, ROWS)
    return _stage_readout(
        h, x, batch3,
        W_fi[:H], W_fi[H:], b_fi.reshape(1, H),
        W_fj[:H], W_fj[H:], b_fj.reshape(1, H),
        W_score.reshape(1, H), b_score.reshape(1, 1),
        W_fc, b_fc.reshape(1, H))


# speedup vs baseline: 16.4319x; 1.2380x over previous
import functools

import jax
import jax.numpy as jnp
from jax import lax
from jax.experimental import pallas as pl
from jax.experimental.pallas import tpu as pltpu
from jax.experimental.pallas import tpu_sc as plsc

N = 10000
E = 320000
F = 128
H = 128
T = 3
B = 16
ITER = 3

ROWS = 400
NBLK = N // ROWS

NC = 2
NS = 16
NW = NC * NS
CH = 64
NCH_MAIN = E // (NW * CH)
NCH = NCH_MAIN + 1
NXW = (E - NW * NCH_MAIN * CH) // CH
NPAD = 10240
STRIP = NPAD // NS

_pallas_call = pl.pallas_call
_f32 = jnp.float32


def _dot(a, b):
    return jnp.dot(a, b, preferred_element_type=_f32)


def _stage_init_body(x_ref, wi_ref, bi_ref, we_ref, be_ref, h_ref, g_ref):
    xb = x_ref[...]
    hb = _dot(xb, wi_ref[...]) + bi_ref[...]
    h_ref[...] = hb
    for t in range(T):
        g_ref[t] = _dot(hb, we_ref[t]) + be_ref[t]


def _stage_init(x, w_init, b_init2, w_edge, b_edge2):
    return _pallas_call(
        _stage_init_body,
        grid=(NBLK,),
        in_specs=[
            pl.BlockSpec((ROWS, F), lambda i: (i, 0)),
            pl.BlockSpec((F, H), lambda i: (0, 0)),
            pl.BlockSpec((1, H), lambda i: (0, 0)),
            pl.BlockSpec((T, H, H), lambda i: (0, 0, 0)),
            pl.BlockSpec((T, H), lambda i: (0, 0)),
        ],
        out_specs=[
            pl.BlockSpec((ROWS, H), lambda i: (i, 0)),
            pl.BlockSpec((T, ROWS, H), lambda i: (0, i, 0)),
        ],
        out_shape=[
            jax.ShapeDtypeStruct((N, H), _f32),
            jax.ShapeDtypeStruct((T, N, H), _f32),
        ],
    )(x, w_init, b_init2, w_edge, b_edge2)


def _stage_gru_body(emit_g, mp_ref, h_ref, wih_ref, whh_ref, bih_ref,
                    bhh_ref, we_ref, be_ref, hn_ref, *maybe_g):
    m = mp_ref[0] + mp_ref[1]
    h = h_ref[...]
    gi = _dot(m, wih_ref[...]) + bih_ref[...]
    gh = _dot(h, whh_ref[...]) + bhh_ref[...]
    r = jax.nn.sigmoid(gi[:, 0:H] + gh[:, 0:H])
    z = jax.nn.sigmoid(gi[:, H:2 * H] + gh[:, H:2 * H])
    n = jnp.tanh(gi[:, 2 * H:] + r * gh[:, 2 * H:])
    hn = (1.0 - z) * n + z * h
    hn_ref[...] = hn
    if emit_g:
        g_ref = maybe_g[0]
        for t in range(T):
            g_ref[t] = _dot(hn, we_ref[t]) + be_ref[t]


def _stage_gru(msg_partial, h, w_ih, w_hh, b_ih2, b_hh2, w_edge, b_edge2,
               emit_g):
    out_specs = [pl.BlockSpec((ROWS, H), lambda i: (i, 0))]
    out_shape = [jax.ShapeDtypeStruct((N, H), _f32)]
    if emit_g:
        out_specs.append(pl.BlockSpec((T, ROWS, H), lambda i: (0, i, 0)))
        out_shape.append(jax.ShapeDtypeStruct((T, N, H), _f32))
    return _pallas_call(
        functools.partial(_stage_gru_body, emit_g),
        grid=(NBLK,),
        in_specs=[
            pl.BlockSpec((NC, ROWS, H), lambda i: (0, i, 0)),
            pl.BlockSpec((ROWS, H), lambda i: (i, 0)),
            pl.BlockSpec((H, 3 * H), lambda i: (0, 0)),
            pl.BlockSpec((H, 3 * H), lambda i: (0, 0)),
            pl.BlockSpec((1, 3 * H), lambda i: (0, 0)),
            pl.BlockSpec((1, 3 * H), lambda i: (0, 0)),
            pl.BlockSpec((T, H, H), lambda i: (0, 0, 0)),
            pl.BlockSpec((T, H), lambda i: (0, 0)),
        ],
        out_specs=out_specs,
        out_shape=out_shape,
    )(msg_partial, h, w_ih, w_hh, b_ih2, b_hh2, w_edge, b_edge2)


def _stage_readout_body(h_ref, x_ref, b_ref, wfih_ref, wfix_ref, bfi_ref,
                        wfjh_ref, wfjx_ref, bfj_ref, ws_ref, bs_ref,
                        wfc_ref, bfc_ref, o_ref, acc_ref):
    i = pl.program_id(0)

    @pl.when(i == 0)
    def _():
        acc_ref[...] = jnp.zeros((B, H), _f32)

    hb = h_ref[...]
    xb = x_ref[...]
    fi = _dot(hb, wfih_ref[...]) + _dot(xb, wfix_ref[...]) + bfi_ref[...]
    fj = jnp.tanh(_dot(hb, wfjh_ref[...]) + _dot(xb, wfjx_ref[...])
                  + bfj_ref[...])
    srow = jnp.sum(fi * ws_ref[...], axis=1, keepdims=True) + bs_ref[0, 0]
    wgt = jax.nn.sigmoid(srow) * fj
    bid = b_ref[0, 0, :]
    onehot = (lax.broadcasted_iota(jnp.int32, (B, ROWS), 0)
              == bid[None, :]).astype(_f32)
    acc_ref[...] += _dot(onehot, wgt)

    @pl.when(i == NBLK - 1)
    def _():
        o_ref[...] = _dot(jnp.tanh(acc_ref[...]), wfc_ref[...]) + bfc_ref[...]


def _stage_readout(h, x, batch3, w_fi_h, w_fi_x, b_fi2, w_fj_h, w_fj_x,
                   b_fj2, ws_row, bs11, w_fc, b_fc2):
    return _pallas_call(
        _stage_readout_body,
        grid=(NBLK,),
        in_specs=[
            pl.BlockSpec((ROWS, H), lambda i: (i, 0)),
            pl.BlockSpec((ROWS, F), lambda i: (i, 0)),
            pl.BlockSpec((1, 1, ROWS), lambda i: (i, 0, 0)),
            pl.BlockSpec((H, H), lambda i: (0, 0)),
            pl.BlockSpec((F, H), lambda i: (0, 0)),
            pl.BlockSpec((1, H), lambda i: (0, 0)),
            pl.BlockSpec((H, H), lambda i: (0, 0)),
            pl.BlockSpec((F, H), lambda i: (0, 0)),
            pl.BlockSpec((1, H), lambda i: (0, 0)),
            pl.BlockSpec((1, H), lambda i: (0, 0)),
            pl.BlockSpec((1, 1), lambda i: (0, 0)),
            pl.BlockSpec((H, H), lambda i: (0, 0)),
            pl.BlockSpec((1, H), lambda i: (0, 0)),
        ],
        out_specs=pl.BlockSpec((B, H), lambda i: (0, 0)),
        out_shape=jax.ShapeDtypeStruct((B, H), _f32),
        scratch_shapes=[pltpu.VMEM((B, H), _f32)],
        compiler_params=pltpu.CompilerParams(
            dimension_semantics=("arbitrary",)),
    )(h, x, batch3, w_fi_h, w_fi_x, b_fi2, w_fj_h, w_fj_x, b_fj2, ws_row,
      bs11, w_fc, b_fc2)


def _sc_body(g_hbm, gidx_hbm, dst_hbm, zeros_hbm, out_hbm,
             gidx_v, dst_v, rows_a, rows_b, acc_sh, sem_a, sem_b):
    cid = lax.axis_index("c")
    sid = lax.axis_index("s")
    wid = sid * NC + cid

    pltpu.sync_copy(zeros_hbm, acc_sh.at[pl.ds(sid * STRIP, STRIP)])

    pltpu.sync_copy(gidx_hbm.at[pl.ds(wid * (NCH * CH), NCH * CH)], gidx_v)
    pltpu.sync_copy(dst_hbm.at[wid], dst_v)
    plsc.subcore_barrier()

    def gat(j, buf, sem):
        return pltpu.make_async_copy(
            g_hbm.at[gidx_v.at[pl.ds(j * CH, CH)]], buf, sem)

    def sca(j, buf):
        pltpu.sync_copy(buf, acc_sh.at[dst_v.at[j]], add=True)

    gat(0, rows_a, sem_a).start()
    gat(1, rows_b, sem_b).start()

    @pl.loop(0, NCH_MAIN - 2, step=2)
    def _(j):
        gat(j, rows_a, sem_a).wait()
        sca(j, rows_a)
        gat(j + 2, rows_a, sem_a).start()
        gat(j + 1, rows_b, sem_b).wait()
        sca(j + 1, rows_b)
        gat(j + 3, rows_b, sem_b).start()

    gat(NCH_MAIN - 2, rows_a, sem_a).wait()
    sca(NCH_MAIN - 2, rows_a)
    gat(NCH_MAIN - 1, rows_b, sem_b).wait()
    sca(NCH_MAIN - 1, rows_b)

    @pl.when(wid < NXW)
    def _():
        pltpu.sync_copy(g_hbm.at[gidx_v.at[pl.ds(NCH_MAIN * CH, CH)]], rows_a)
        sca(NCH_MAIN, rows_a)

    plsc.subcore_barrier()
    pltpu.sync_copy(acc_sh.at[pl.ds(sid * STRIP, STRIP)],
                    out_hbm.at[cid, pl.ds(sid * STRIP, STRIP)])


def _sc_messages(g_table, gidx_r, dst_r, zeros):
    mesh = plsc.VectorSubcoreMesh(core_axis_name="c", subcore_axis_name="s")
    k = pl.kernel(
        _sc_body,
        mesh=mesh,
        out_type=jax.ShapeDtypeStruct((NC, NPAD, H), _f32),
        scratch_types=[
            pltpu.VMEM((NCH * CH,), jnp.int32),
            pltpu.VMEM((NCH, CH), jnp.int32),
            pltpu.VMEM((CH, H), _f32),
            pltpu.VMEM((CH, H), _f32),
            pltpu.VMEM_SHARED((NPAD, H), _f32),
            pltpu.SemaphoreType.DMA,
            pltpu.SemaphoreType.DMA,
        ],
    )
    return k(g_table, gidx_r, dst_r, zeros)


def kernel(x, edge_index, edge_attr, batch, W_init, b_init, W_edge, b_edge,
           bias, W_ih, W_hh, b_ih, b_hh, W_fi, b_fi, W_fj, b_fj, W_score,
           b_score, W_fc, b_fc):
    src = edge_index[0]
    dst = edge_index[1]

    gidx = edge_attr.astype(jnp.int32) * N + src
    e_main = NW * NCH_MAIN * CH
    gidx_main = gidx[:e_main].reshape(NW, NCH_MAIN * CH)
    gidx_x = jnp.zeros((NW, CH), jnp.int32).at[:NXW].set(
        gidx[e_main:].reshape(NXW, CH))
    gidx_r = jnp.concatenate([gidx_main, gidx_x], axis=1).reshape(-1)
    dst_main = dst[:e_main].reshape(NW, NCH_MAIN, CH)
    dst_x = jnp.zeros((NW, 1, CH), jnp.int32).at[:NXW].set(
        dst[e_main:].reshape(NXW, 1, CH))
    dst_r = jnp.concatenate([dst_main, dst_x], axis=1)
    zeros = jnp.zeros((STRIP, H), _f32)

    b_init2 = b_init.reshape(1, H)
    b_edge2 = b_edge + bias[None, :]
    b_ih2 = b_ih.reshape(1, 3 * H)
    b_hh2 = b_hh.reshape(1, 3 * H)

    h, g = _stage_init(x, W_init, b_init2, W_edge, b_edge2)
    for it in range(ITER):
        msg_partial = _sc_messages(g.reshape(T * N, H), gidx_r, dst_r, zeros)
        emit_g = it < ITER - 1
        outs = _stage_gru(msg_partial, h, W_ih, W_hh, b_ih2, b_hh2, W_edge,
                          b_edge2, emit_g)
        if emit_g:
            h, g = outs
        else:
            h, = outs

    batch3 = batch.reshape(NBLK, 1, ROWS)
    return _stage_readout(
        h, x, batch3,
        W_fi[:H], W_fi[H:], b_fi.reshape(1, H),
        W_fj[:H], W_fj[H:], b_fj.reshape(1, H),
        W_score.reshape(1, H), b_score.reshape(1, 1),
        W_fc, b_fc.reshape(1, H))


# speedup vs baseline: 16.6117x; 1.0109x over previous
import functools

import jax
import jax.numpy as jnp
from jax import lax
from jax.experimental import pallas as pl
from jax.experimental.pallas import tpu as pltpu
from jax.experimental.pallas import tpu_sc as plsc

N = 10000
E = 320000
F = 128
H = 128
T = 3
B = 16
ITER = 3

ROWS = 400
NBLK = N // ROWS

NC = 2
NS = 16
NW = NC * NS
CH = 64
NCH_MAIN = E // (NW * CH)
NCH = NCH_MAIN + 1
NXW = (E - NW * NCH_MAIN * CH) // CH
NPAD = 10240
STRIP = NPAD // NS

_pallas_call = pl.pallas_call
_f32 = jnp.float32


def _dot(a, b):
    return jnp.dot(a, b, preferred_element_type=_f32)


def _stage_init_body(x_ref, wi_ref, bi_ref, we_ref, be_ref, h_ref, g_ref):
    xb = x_ref[...]
    hb = _dot(xb, wi_ref[...]) + bi_ref[...]
    h_ref[...] = hb
    for t in range(T):
        g_ref[t] = _dot(hb, we_ref[t]) + be_ref[t]


def _stage_init(x, w_init, b_init2, w_edge, b_edge2):
    return _pallas_call(
        _stage_init_body,
        grid=(NBLK,),
        in_specs=[
            pl.BlockSpec((ROWS, F), lambda i: (i, 0)),
            pl.BlockSpec((F, H), lambda i: (0, 0)),
            pl.BlockSpec((1, H), lambda i: (0, 0)),
            pl.BlockSpec((T, H, H), lambda i: (0, 0, 0)),
            pl.BlockSpec((T, H), lambda i: (0, 0)),
        ],
        out_specs=[
            pl.BlockSpec((ROWS, H), lambda i: (i, 0)),
            pl.BlockSpec((T, ROWS, H), lambda i: (0, i, 0)),
        ],
        out_shape=[
            jax.ShapeDtypeStruct((N, H), _f32),
            jax.ShapeDtypeStruct((T, N, H), _f32),
        ],
    )(x, w_init, b_init2, w_edge, b_edge2)


def _stage_gru_body(emit_g, mp_ref, h_ref, wih_ref, whh_ref, bih_ref,
                    bhh_ref, we_ref, be_ref, hn_ref, *maybe_g):
    m = mp_ref[0] + mp_ref[1]
    h = h_ref[...]
    gi = _dot(m, wih_ref[...]) + bih_ref[...]
    gh = _dot(h, whh_ref[...]) + bhh_ref[...]
    r = jax.nn.sigmoid(gi[:, 0:H] + gh[:, 0:H])
    z = jax.nn.sigmoid(gi[:, H:2 * H] + gh[:, H:2 * H])
    n = jnp.tanh(gi[:, 2 * H:] + r * gh[:, 2 * H:])
    hn = (1.0 - z) * n + z * h
    hn_ref[...] = hn
    if emit_g:
        g_ref = maybe_g[0]
        for t in range(T):
            g_ref[t] = _dot(hn, we_ref[t]) + be_ref[t]


def _stage_gru(msg_partial, h, w_ih, w_hh, b_ih2, b_hh2, w_edge, b_edge2,
               emit_g):
    out_specs = [pl.BlockSpec((ROWS, H), lambda i: (i, 0))]
    out_shape = [jax.ShapeDtypeStruct((N, H), _f32)]
    if emit_g:
        out_specs.append(pl.BlockSpec((T, ROWS, H), lambda i: (0, i, 0)))
        out_shape.append(jax.ShapeDtypeStruct((T, N, H), _f32))
    return _pallas_call(
        functools.partial(_stage_gru_body, emit_g),
        grid=(NBLK,),
        in_specs=[
            pl.BlockSpec((NC, ROWS, H), lambda i: (0, i, 0)),
            pl.BlockSpec((ROWS, H), lambda i: (i, 0)),
            pl.BlockSpec((H, 3 * H), lambda i: (0, 0)),
            pl.BlockSpec((H, 3 * H), lambda i: (0, 0)),
            pl.BlockSpec((1, 3 * H), lambda i: (0, 0)),
            pl.BlockSpec((1, 3 * H), lambda i: (0, 0)),
            pl.BlockSpec((T, H, H), lambda i: (0, 0, 0)),
            pl.BlockSpec((T, H), lambda i: (0, 0)),
        ],
        out_specs=out_specs,
        out_shape=out_shape,
    )(msg_partial, h, w_ih, w_hh, b_ih2, b_hh2, w_edge, b_edge2)


def _stage_readout_body(h_ref, x_ref, b_ref, wfih_ref, wfix_ref, bfi_ref,
                        wfjh_ref, wfjx_ref, bfj_ref, ws_ref, bs_ref,
                        wfc_ref, bfc_ref, o_ref, acc_ref):
    i = pl.program_id(0)

    @pl.when(i == 0)
    def _():
        acc_ref[...] = jnp.zeros((B, H), _f32)

    hb = h_ref[...]
    xb = x_ref[...]
    fi = _dot(hb, wfih_ref[...]) + _dot(xb, wfix_ref[...]) + bfi_ref[...]
    fj = jnp.tanh(_dot(hb, wfjh_ref[...]) + _dot(xb, wfjx_ref[...])
                  + bfj_ref[...])
    srow = jnp.sum(fi * ws_ref[...], axis=1, keepdims=True) + bs_ref[0, 0]
    wgt = jax.nn.sigmoid(srow) * fj
    bid = b_ref[0, 0, :]
    onehot = (lax.broadcasted_iota(jnp.int32, (B, ROWS), 0)
              == bid[None, :]).astype(_f32)
    acc_ref[...] += _dot(onehot, wgt)

    @pl.when(i == NBLK - 1)
    def _():
        o_ref[...] = _dot(jnp.tanh(acc_ref[...]), wfc_ref[...]) + bfc_ref[...]


def _stage_readout(h, x, batch3, w_fi_h, w_fi_x, b_fi2, w_fj_h, w_fj_x,
                   b_fj2, ws_row, bs11, w_fc, b_fc2):
    return _pallas_call(
        _stage_readout_body,
        grid=(NBLK,),
        in_specs=[
            pl.BlockSpec((ROWS, H), lambda i: (i, 0)),
            pl.BlockSpec((ROWS, F), lambda i: (i, 0)),
            pl.BlockSpec((1, 1, ROWS), lambda i: (i, 0, 0)),
            pl.BlockSpec((H, H), lambda i: (0, 0)),
            pl.BlockSpec((F, H), lambda i: (0, 0)),
            pl.BlockSpec((1, H), lambda i: (0, 0)),
            pl.BlockSpec((H, H), lambda i: (0, 0)),
            pl.BlockSpec((F, H), lambda i: (0, 0)),
            pl.BlockSpec((1, H), lambda i: (0, 0)),
            pl.BlockSpec((1, H), lambda i: (0, 0)),
            pl.BlockSpec((1, 1), lambda i: (0, 0)),
            pl.BlockSpec((H, H), lambda i: (0, 0)),
            pl.BlockSpec((1, H), lambda i: (0, 0)),
        ],
        out_specs=pl.BlockSpec((B, H), lambda i: (0, 0)),
        out_shape=jax.ShapeDtypeStruct((B, H), _f32),
        scratch_shapes=[pltpu.VMEM((B, H), _f32)],
        compiler_params=pltpu.CompilerParams(
            dimension_semantics=("arbitrary",)),
    )(h, x, batch3, w_fi_h, w_fi_x, b_fi2, w_fj_h, w_fj_x, b_fj2, ws_row,
      bs11, w_fc, b_fc2)


def _sc_body(g_hbm, gidx_hbm, dst_hbm, zeros_hbm, out_hbm,
             gidx_v, dst_v, rows_a, rows_b, acc_sh,
             sem_a, sem_b, sem_p):
    cid = lax.axis_index("c")
    sid = lax.axis_index("s")
    wid = sid * NC + cid

    z = pltpu.make_async_copy(zeros_hbm, acc_sh.at[pl.ds(sid * STRIP, STRIP)],
                              sem_p)
    gi = pltpu.make_async_copy(
        gidx_hbm.at[pl.ds(wid * (NCH * CH), NCH * CH)], gidx_v, sem_p)
    di = pltpu.make_async_copy(dst_hbm.at[wid], dst_v, sem_p)
    z.start()
    gi.start()
    di.start()
    z.wait()
    gi.wait()
    di.wait()
    plsc.subcore_barrier()

    def gat(j, buf, sem):
        return pltpu.make_async_copy(
            g_hbm.at[gidx_v.at[pl.ds(j * CH, CH)]], buf, sem)

    def sca(j, buf):
        pltpu.sync_copy(buf, acc_sh.at[dst_v.at[j]], add=True)

    gat(0, rows_a, sem_a).start()
    gat(1, rows_b, sem_b).start()

    @pl.loop(0, NCH_MAIN - 2, step=2)
    def _(j):
        gat(j, rows_a, sem_a).wait()
        sca(j, rows_a)
        gat(j + 2, rows_a, sem_a).start()
        gat(j + 1, rows_b, sem_b).wait()
        sca(j + 1, rows_b)
        gat(j + 3, rows_b, sem_b).start()

    gat(NCH_MAIN - 2, rows_a, sem_a).wait()
    sca(NCH_MAIN - 2, rows_a)
    gat(NCH_MAIN - 1, rows_b, sem_b).wait()
    sca(NCH_MAIN - 1, rows_b)

    @pl.when(wid < NXW)
    def _():
        pltpu.sync_copy(g_hbm.at[gidx_v.at[pl.ds(NCH_MAIN * CH, CH)]], rows_a)
        sca(NCH_MAIN, rows_a)

    plsc.subcore_barrier()
    pltpu.sync_copy(acc_sh.at[pl.ds(sid * STRIP, STRIP)],
                    out_hbm.at[cid, pl.ds(sid * STRIP, STRIP)])


def _sc_messages(g_table, gidx_r, dst_r, zeros):
    mesh = plsc.VectorSubcoreMesh(core_axis_name="c", subcore_axis_name="s")
    k = pl.kernel(
        _sc_body,
        mesh=mesh,
        out_type=jax.ShapeDtypeStruct((NC, NPAD, H), _f32),
        scratch_types=[
            pltpu.VMEM((NCH * CH,), jnp.int32),
            pltpu.VMEM((NCH, CH), jnp.int32),
            pltpu.VMEM((CH, H), _f32),
            pltpu.VMEM((CH, H), _f32),
            pltpu.VMEM_SHARED((NPAD, H), _f32),
            pltpu.SemaphoreType.DMA,
            pltpu.SemaphoreType.DMA,
            pltpu.SemaphoreType.DMA,
        ],
    )
    return k(g_table, gidx_r, dst_r, zeros)


def kernel(x, edge_index, edge_attr, batch, W_init, b_init, W_edge, b_edge,
           bias, W_ih, W_hh, b_ih, b_hh, W_fi, b_fi, W_fj, b_fj, W_score,
           b_score, W_fc, b_fc):
    src = edge_index[0]
    dst = edge_index[1]

    gidx = edge_attr.astype(jnp.int32) * N + src
    e_main = NW * NCH_MAIN * CH
    gidx_main = gidx[:e_main].reshape(NW, NCH_MAIN * CH)
    gidx_x = jnp.zeros((NW, CH), jnp.int32).at[:NXW].set(
        gidx[e_main:].reshape(NXW, CH))
    gidx_r = jnp.concatenate([gidx_main, gidx_x], axis=1).reshape(-1)
    dst_main = dst[:e_main].reshape(NW, NCH_MAIN, CH)
    dst_x = jnp.zeros((NW, 1, CH), jnp.int32).at[:NXW].set(
        dst[e_main:].reshape(NXW, 1, CH))
    dst_r = jnp.concatenate([dst_main, dst_x], axis=1)
    zeros = jnp.zeros((STRIP, H), _f32)

    b_init2 = b_init.reshape(1, H)
    b_edge2 = b_edge + bias[None, :]
    b_ih2 = b_ih.reshape(1, 3 * H)
    b_hh2 = b_hh.reshape(1, 3 * H)

    h, g = _stage_init(x, W_init, b_init2, W_edge, b_edge2)
    for it in range(ITER):
        msg_partial = _sc_messages(g.reshape(T * N, H), gidx_r, dst_r, zeros)
        emit_g = it < ITER - 1
        outs = _stage_gru(msg_partial, h, W_ih, W_hh, b_ih2, b_hh2, W_edge,
                          b_edge2, emit_g)
        if emit_g:
            h, g = outs
        else:
            h, = outs

    batch3 = batch.reshape(NBLK, 1, ROWS)
    return _stage_readout(
        h, x, batch3,
        W_fi[:H], W_fi[H:], b_fi.reshape(1, H),
        W_fj[:H], W_fj[H:], b_fj.reshape(1, H),
        W_score.reshape(1, H), b_score.reshape(1, 1),
        W_fc, b_fc.reshape(1, H))


# speedup vs baseline: 17.1169x; 1.0304x over previous
import functools

import jax
import jax.numpy as jnp
from jax import lax
from jax.experimental import pallas as pl
from jax.experimental.pallas import tpu as pltpu
from jax.experimental.pallas import tpu_sc as plsc

N = 10000
E = 320000
F = 128
H = 128
T = 3
B = 16
ITER = 3

ROWS = 400
NBLK = N // ROWS

NC = 2
NS = 16
NW = NC * NS
CH = 64
NCH_MAIN = E // (NW * CH)
NCH = NCH_MAIN + 1
NXW = (E - NW * NCH_MAIN * CH) // CH
NPAD = 10240
STRIP = NPAD // NS

_pallas_call = pl.pallas_call
_f32 = jnp.float32


_bf16 = jnp.bfloat16


def _dot(a, b):
    return jnp.dot(a, b, preferred_element_type=_f32)


_PARALLEL = pltpu.CompilerParams(dimension_semantics=("parallel",))
_ARBITRARY = pltpu.CompilerParams(dimension_semantics=("arbitrary",))


def _stage_init_body(x_ref, wi_ref, bi_ref, we_ref, be_ref, h_ref, g_ref):
    xb = x_ref[...]
    hb = _dot(xb, wi_ref[...]) + bi_ref[...]
    h_ref[...] = hb
    for t in range(T):
        g_ref[t] = _dot(hb, we_ref[t]) + be_ref[t]


def _stage_init(x, w_init, b_init2, w_edge, b_edge2):
    return _pallas_call(
        _stage_init_body,
        grid=(NBLK,),
        in_specs=[
            pl.BlockSpec((ROWS, F), lambda i: (i, 0)),
            pl.BlockSpec((F, H), lambda i: (0, 0)),
            pl.BlockSpec((1, H), lambda i: (0, 0)),
            pl.BlockSpec((T, H, H), lambda i: (0, 0, 0)),
            pl.BlockSpec((T, H), lambda i: (0, 0)),
        ],
        out_specs=[
            pl.BlockSpec((ROWS, H), lambda i: (i, 0)),
            pl.BlockSpec((T, ROWS, H), lambda i: (0, i, 0)),
        ],
        out_shape=[
            jax.ShapeDtypeStruct((N, H), _f32),
            jax.ShapeDtypeStruct((T, N, H), _f32),
        ],
        compiler_params=_PARALLEL,
    )(x, w_init, b_init2, w_edge, b_edge2)


def _gru_block(mp_ref, h_ref, wih_ref, whh_ref, bih_ref, bhh_ref):
    m = mp_ref[0] + mp_ref[1]
    h = h_ref[...]
    gi = _dot(m, wih_ref[...]) + bih_ref[...]
    gh = _dot(h, whh_ref[...]) + bhh_ref[...]
    r = jax.nn.sigmoid(gi[:, 0:H] + gh[:, 0:H])
    z = jax.nn.sigmoid(gi[:, H:2 * H] + gh[:, H:2 * H])
    n = jnp.tanh(gi[:, 2 * H:] + r * gh[:, 2 * H:])
    return (1.0 - z) * n + z * h


def _stage_gru_body(mp_ref, h_ref, wih_ref, whh_ref, bih_ref,
                    bhh_ref, we_ref, be_ref, hn_ref, g_ref):
    hn = _gru_block(mp_ref, h_ref, wih_ref, whh_ref, bih_ref, bhh_ref)
    hn_ref[...] = hn
    for t in range(T):
        g_ref[t] = _dot(hn, we_ref[t]) + be_ref[t]


def _stage_gru(msg_partial, h, w_ih, w_hh, b_ih2, b_hh2, w_edge, b_edge2):
    return _pallas_call(
        _stage_gru_body,
        grid=(NBLK,),
        in_specs=[
            pl.BlockSpec((NC, ROWS, H), lambda i: (0, i, 0)),
            pl.BlockSpec((ROWS, H), lambda i: (i, 0)),
            pl.BlockSpec((H, 3 * H), lambda i: (0, 0)),
            pl.BlockSpec((H, 3 * H), lambda i: (0, 0)),
            pl.BlockSpec((1, 3 * H), lambda i: (0, 0)),
            pl.BlockSpec((1, 3 * H), lambda i: (0, 0)),
            pl.BlockSpec((T, H, H), lambda i: (0, 0, 0)),
            pl.BlockSpec((T, H), lambda i: (0, 0)),
        ],
        out_specs=[
            pl.BlockSpec((ROWS, H), lambda i: (i, 0)),
            pl.BlockSpec((T, ROWS, H), lambda i: (0, i, 0)),
        ],
        out_shape=[
            jax.ShapeDtypeStruct((N, H), _f32),
            jax.ShapeDtypeStruct((T, N, H), _f32),
        ],
        compiler_params=_PARALLEL,
    )(msg_partial, h, w_ih, w_hh, b_ih2, b_hh2, w_edge, b_edge2)


def _stage_final_body(mp_ref, h_ref, wih_ref, whh_ref, bih_ref, bhh_ref,
                      x_ref, b_ref, wfih_ref, wfix_ref, bfi_ref,
                      wfjh_ref, wfjx_ref, bfj_ref, ws_ref, bs_ref,
                      wfc_ref, bfc_ref, o_ref, acc_ref):
    i = pl.program_id(0)

    @pl.when(i == 0)
    def _():
        acc_ref[...] = jnp.zeros((B, H), _f32)

    hb = _gru_block(mp_ref, h_ref, wih_ref, whh_ref, bih_ref, bhh_ref)
    xb = x_ref[...]
    fi = _dot(hb, wfih_ref[...]) + _dot(xb, wfix_ref[...]) + bfi_ref[...]
    fj = jnp.tanh(_dot(hb, wfjh_ref[...]) + _dot(xb, wfjx_ref[...])
                  + bfj_ref[...])
    srow = jnp.sum(fi * ws_ref[...], axis=1, keepdims=True) + bs_ref[0, 0]
    wgt = jax.nn.sigmoid(srow) * fj
    bid = b_ref[0, 0, :]
    onehot = (lax.broadcasted_iota(jnp.int32, (B, ROWS), 0)
              == bid[None, :]).astype(_f32)
    acc_ref[...] += _dot(onehot, wgt)

    @pl.when(i == NBLK - 1)
    def _():
        o_ref[...] = _dot(jnp.tanh(acc_ref[...]), wfc_ref[...]) + bfc_ref[...]


def _stage_final(msg_partial, h, w_ih, w_hh, b_ih2, b_hh2, x, batch3,
                 w_fi_h, w_fi_x, b_fi2, w_fj_h, w_fj_x, b_fj2, ws_row,
                 bs11, w_fc, b_fc2):
    return _pallas_call(
        _stage_final_body,
        grid=(NBLK,),
        in_specs=[
            pl.BlockSpec((NC, ROWS, H), lambda i: (0, i, 0)),
            pl.BlockSpec((ROWS, H), lambda i: (i, 0)),
            pl.BlockSpec((H, 3 * H), lambda i: (0, 0)),
            pl.BlockSpec((H, 3 * H), lambda i: (0, 0)),
            pl.BlockSpec((1, 3 * H), lambda i: (0, 0)),
            pl.BlockSpec((1, 3 * H), lambda i: (0, 0)),
            pl.BlockSpec((ROWS, F), lambda i: (i, 0)),
            pl.BlockSpec((1, 1, ROWS), lambda i: (i, 0, 0)),
            pl.BlockSpec((H, H), lambda i: (0, 0)),
            pl.BlockSpec((F, H), lambda i: (0, 0)),
            pl.BlockSpec((1, H), lambda i: (0, 0)),
            pl.BlockSpec((H, H), lambda i: (0, 0)),
            pl.BlockSpec((F, H), lambda i: (0, 0)),
            pl.BlockSpec((1, H), lambda i: (0, 0)),
            pl.BlockSpec((1, H), lambda i: (0, 0)),
            pl.BlockSpec((1, 1), lambda i: (0, 0)),
            pl.BlockSpec((H, H), lambda i: (0, 0)),
            pl.BlockSpec((1, H), lambda i: (0, 0)),
        ],
        out_specs=pl.BlockSpec((B, H), lambda i: (0, 0)),
        out_shape=jax.ShapeDtypeStruct((B, H), _f32),
        scratch_shapes=[pltpu.VMEM((B, H), _f32)],
        compiler_params=_ARBITRARY,
    )(msg_partial, h, w_ih, w_hh, b_ih2, b_hh2, x, batch3, w_fi_h, w_fi_x,
      b_fi2, w_fj_h, w_fj_x, b_fj2, ws_row, bs11, w_fc, b_fc2)


def _sc_body(g_hbm, gidx_hbm, dst_hbm, zeros_hbm, out_hbm,
             gidx_v, dst_v, rows_a, rows_b, acc_sh,
             sem_a, sem_b, sem_p):
    cid = lax.axis_index("c")
    sid = lax.axis_index("s")
    wid = sid * NC + cid

    z = pltpu.make_async_copy(zeros_hbm, acc_sh.at[pl.ds(sid * STRIP, STRIP)],
                              sem_p)
    gi = pltpu.make_async_copy(
        gidx_hbm.at[pl.ds(wid * (NCH * CH), NCH * CH)], gidx_v, sem_p)
    di = pltpu.make_async_copy(dst_hbm.at[wid], dst_v, sem_p)
    z.start()
    gi.start()
    di.start()
    z.wait()
    gi.wait()
    di.wait()
    plsc.subcore_barrier()

    def gat(j, buf, sem):
        return pltpu.make_async_copy(
            g_hbm.at[gidx_v.at[pl.ds(j * CH, CH)]], buf, sem)

    def sca(j, buf):
        pltpu.sync_copy(buf, acc_sh.at[dst_v.at[j]], add=True)

    gat(0, rows_a, sem_a).start()
    gat(1, rows_b, sem_b).start()

    @pl.loop(0, NCH_MAIN - 2, step=2)
    def _(j):
        gat(j, rows_a, sem_a).wait()
        sca(j, rows_a)
        gat(j + 2, rows_a, sem_a).start()
        gat(j + 1, rows_b, sem_b).wait()
        sca(j + 1, rows_b)
        gat(j + 3, rows_b, sem_b).start()

    gat(NCH_MAIN - 2, rows_a, sem_a).wait()
    sca(NCH_MAIN - 2, rows_a)
    gat(NCH_MAIN - 1, rows_b, sem_b).wait()
    sca(NCH_MAIN - 1, rows_b)

    @pl.when(wid < NXW)
    def _():
        pltpu.sync_copy(g_hbm.at[gidx_v.at[pl.ds(NCH_MAIN * CH, CH)]], rows_a)
        sca(NCH_MAIN, rows_a)

    plsc.subcore_barrier()
    pltpu.sync_copy(acc_sh.at[pl.ds(sid * STRIP, STRIP)],
                    out_hbm.at[cid, pl.ds(sid * STRIP, STRIP)])


def _sc_messages(g_table, gidx_r, dst_r, zeros):
    mesh = plsc.VectorSubcoreMesh(core_axis_name="c", subcore_axis_name="s")
    k = pl.kernel(
        _sc_body,
        mesh=mesh,
        out_type=jax.ShapeDtypeStruct((NC, NPAD, H), _f32),
        scratch_types=[
            pltpu.VMEM((NCH * CH,), jnp.int32),
            pltpu.VMEM((NCH, CH), jnp.int32),
            pltpu.VMEM((CH, H), _f32),
            pltpu.VMEM((CH, H), _f32),
            pltpu.VMEM_SHARED((NPAD, H), _f32),
            pltpu.SemaphoreType.DMA,
            pltpu.SemaphoreType.DMA,
            pltpu.SemaphoreType.DMA,
        ],
    )
    return k(g_table, gidx_r, dst_r, zeros)


def kernel(x, edge_index, edge_attr, batch, W_init, b_init, W_edge, b_edge,
           bias, W_ih, W_hh, b_ih, b_hh, W_fi, b_fi, W_fj, b_fj, W_score,
           b_score, W_fc, b_fc):
    src = edge_index[0]
    dst = edge_index[1]

    gidx = edge_attr.astype(jnp.int32) * N + src
    e_main = NW * NCH_MAIN * CH
    gidx_main = gidx[:e_main].reshape(NW, NCH_MAIN * CH)
    gidx_x = jnp.zeros((NW, CH), jnp.int32).at[:NXW].set(
        gidx[e_main:].reshape(NXW, CH))
    gidx_r = jnp.concatenate([gidx_main, gidx_x], axis=1).reshape(-1)
    dst_main = dst[:e_main].reshape(NW, NCH_MAIN, CH)
    dst_x = jnp.zeros((NW, 1, CH), jnp.int32).at[:NXW].set(
        dst[e_main:].reshape(NXW, 1, CH))
    dst_r = jnp.concatenate([dst_main, dst_x], axis=1)
    zeros = jnp.zeros((STRIP, H), _f32)

    b_init2 = b_init.reshape(1, H)
    b_edge2 = b_edge + bias[None, :]
    b_ih2 = b_ih.reshape(1, 3 * H)
    b_hh2 = b_hh.reshape(1, 3 * H)

    wi_b = W_init
    we_b = W_edge
    wih_b = W_ih
    whh_b = W_hh

    h, g = _stage_init(x, wi_b, b_init2, we_b, b_edge2)
    for it in range(ITER - 1):
        msg_partial = _sc_messages(g.reshape(T * N, H), gidx_r, dst_r, zeros)
        h, g = _stage_gru(msg_partial, h, wih_b, whh_b, b_ih2, b_hh2, we_b,
                          b_edge2)

    msg_partial = _sc_messages(g.reshape(T * N, H), gidx_r, dst_r, zeros)
    batch3 = batch.reshape(NBLK, 1, ROWS)
    return _stage_final(
        msg_partial, h, wih_b, whh_b, b_ih2, b_hh2, x, batch3,
        W_fi[:H], W_fi[H:], b_fi.reshape(1, H),
        W_fj[:H], W_fj[H:], b_fj.reshape(1, H),
        W_score.reshape(1, H), b_score.reshape(1, 1),
        W_fc, b_fc.reshape(1, H))
